# baseline (speedup 1.0000x reference)
import functools

import jax
import jax.numpy as jnp
from jax import lax
from jax.experimental import pallas as pl
from jax.experimental.pallas import tpu as pltpu
from jax.experimental.pallas import tpu_sc as plsc

_NPROP = 128
_RADIUS = 0.3
_K = 16
_INF = float("inf")


def _fps_kernel(xs_ref, ys_ref, nx_ref, ny_ref):
    xv = xs_ref[...]
    yv = ys_ref[...]
    B, V = xv.shape
    colV = lax.broadcasted_iota(jnp.int32, (B, V), 1)
    colP = lax.broadcasted_iota(jnp.int32, (B, _NPROP), 1)

    def step(i, carry):
        dist, far, nx, ny = carry
        sel = colV == far
        cx = jnp.sum(jnp.where(sel, xv, 0.0), axis=1, keepdims=True)
        cy = jnp.sum(jnp.where(sel, yv, 0.0), axis=1, keepdims=True)
        emit = colP == i
        nx = jnp.where(emit, cx, nx)
        ny = jnp.where(emit, cy, ny)
        d = (xv - cx) ** 2 + (yv - cy) ** 2
        dist = jnp.minimum(dist, d)
        m = jnp.max(dist, axis=1, keepdims=True)
        far = jnp.min(jnp.where(dist == m, colV, V), axis=1, keepdims=True)
        return dist, far, nx, ny

    dist0 = jnp.full((B, V), 1e10, dtype=jnp.float32)
    zeros = jnp.zeros((B, _NPROP), dtype=jnp.float32)
    _, _, nx, ny = lax.fori_loop(
        0, _NPROP, step, (dist0, jnp.zeros((B, 1), jnp.int32), zeros, zeros))
    nx_ref[...] = nx[:, None, :]
    ny_ref[...] = ny[:, None, :]


def _run_fps(xs, ys):
    B, V = xs.shape
    return pl.pallas_call(
        _fps_kernel,
        out_shape=[
            jax.ShapeDtypeStruct((B, 1, _NPROP), jnp.float32),
            jax.ShapeDtypeStruct((B, 1, _NPROP), jnp.float32),
        ],
    )(xs, ys)


def _bq_kernel(xyz_ref, nx_ref, ny_ref, idx_ref, gx_ref, gy_ref):
    xv = xyz_ref[0, 0:1, :]
    yv = xyz_ref[0, 1:2, :]
    V = xv.shape[1]
    nx = jnp.transpose(nx_ref[0])
    ny = jnp.transpose(ny_ref[0])
    d = jnp.sqrt((nx - xv) ** 2 + (ny - yv) ** 2)
    col = lax.broadcasted_iota(jnp.int32, (_NPROP, V), 1)
    x0 = jnp.sum(xv[:, 0:1])
    y0 = jnp.sum(yv[:, 0:1])

    m = None
    idxp = None
    for k in range(_K):
        if k == 0:
            d_eff = d
        else:
            live = (d > m) | ((d == m) & (col > idxp[:, None]))
            d_eff = jnp.where(live, d, _INF)
        m = jnp.min(d_eff, axis=1, keepdims=True)
        idxk = jnp.min(jnp.where(d_eff == m, col, V), axis=1)
        idxp = idxk
        onehot = col == idxk[:, None]
        xk = jnp.sum(jnp.where(onehot, xv, 0.0), axis=1)
        yk = jnp.sum(jnp.where(onehot, yv, 0.0), axis=1)
        inr = m[:, 0] <= _RADIUS
        idx_ref[0, :, k] = jnp.where(inr, idxk, 0)
        gx_ref[0, :, k] = jnp.where(inr, xk, x0) - nx[:, 0]
        gy_ref[0, :, k] = jnp.where(inr, yk, y0) - ny[:, 0]


def _run_ballquery(xyz_t, nx, ny):
    B, _, V = xyz_t.shape
    return pl.pallas_call(
        _bq_kernel,
        grid=(B,),
        in_specs=[
            pl.BlockSpec((1, 2, V), lambda b: (b, 0, 0)),
            pl.BlockSpec((1, 1, _NPROP), lambda b: (b, 0, 0)),
            pl.BlockSpec((1, 1, _NPROP), lambda b: (b, 0, 0)),
        ],
        out_specs=[
            pl.BlockSpec((1, _NPROP, _K), lambda b: (b, 0, 0)),
            pl.BlockSpec((1, _NPROP, _K), lambda b: (b, 0, 0)),
            pl.BlockSpec((1, _NPROP, _K), lambda b: (b, 0, 0)),
        ],
        out_shape=[
            jax.ShapeDtypeStruct((B, _NPROP, _K), jnp.int32),
            jax.ShapeDtypeStruct((B, _NPROP, _K), jnp.float32),
            jax.ShapeDtypeStruct((B, _NPROP, _K), jnp.float32),
        ],
    )(xyz_t, nx, ny)


@functools.lru_cache(maxsize=None)
def _make_sc_gather(n_rows, feat):
    info = plsc.get_sparse_core_info()
    nw = info.num_cores * info.num_subcores
    rows_per_w = n_rows // nw
    mesh = plsc.VectorSubcoreMesh(core_axis_name="c", subcore_axis_name="s")

    @functools.partial(
        pl.kernel,
        mesh=mesh,
        out_type=jax.ShapeDtypeStruct((n_rows, feat), jnp.float32),
        scratch_types=[
            pltpu.VMEM((rows_per_w,), jnp.int32),
            pltpu.VMEM((rows_per_w, feat), jnp.float32),
            pltpu.SemaphoreType.DMA,
        ],
    )
    def gather_k(table_hbm, idx_hbm, out_hbm, idx_v, rows_v, sem):
        wid = lax.axis_index("s") * info.num_cores + lax.axis_index("c")
        base = wid * rows_per_w
        pltpu.sync_copy(idx_hbm.at[pl.ds(base, rows_per_w)], idx_v)
        pltpu.async_copy(table_hbm.at[idx_v], rows_v, sem).wait()
        pltpu.sync_copy(rows_v, out_hbm.at[pl.ds(base, rows_per_w)])

    return gather_k


def _run_sc_gather(table, idx_flat):
    return _make_sc_gather(idx_flat.shape[0], table.shape[1])(table, idx_flat)


def _mlp_kernel(feats_ref, gxy_ref, w1f_ref, w1xy_ref, b1_ref, g1_ref,
                be1_ref, w2_ref, b2_ref, g2_ref, be2_ref, out_ref):
    n = feats_ref.shape[0]
    h1 = (jnp.dot(feats_ref[...], w1f_ref[...],
                  preferred_element_type=jnp.float32)
          + jnp.dot(gxy_ref[...], w1xy_ref[...],
                    preferred_element_type=jnp.float32)
          + b1_ref[...])
    mu = jnp.mean(h1, axis=0, keepdims=True)
    var = jnp.mean((h1 - mu) ** 2, axis=0, keepdims=True)
    a1 = (h1 - mu) * lax.rsqrt(var + 1e-5) * g1_ref[...] + be1_ref[...]
    a1 = jnp.maximum(a1, 0.0)
    h2 = jnp.dot(a1, w2_ref[...], preferred_element_type=jnp.float32) + b2_ref[...]
    mu2 = jnp.mean(h2, axis=0, keepdims=True)
    var2 = jnp.mean((h2 - mu2) ** 2, axis=0, keepdims=True)
    a2 = (h2 - mu2) * lax.rsqrt(var2 + 1e-5) * g2_ref[...] + be2_ref[...]
    a2 = jnp.maximum(a2, 0.0)
    pooled = jnp.max(a2.reshape(n // _K, _K, a2.shape[1]), axis=1)
    out_ref[...] = pooled


def _run_mlp(feats_g, gxy, w1f, w1xy, b1, g1, be1, w2t, b2, g2, be2):
    n, cf = feats_g.shape
    co = w2t.shape[1]
    return pl.pallas_call(
        _mlp_kernel,
        out_shape=jax.ShapeDtypeStruct((n // _K, co), jnp.float32),
    )(feats_g, gxy, w1f, w1xy, b1, g1, be1, w2t, b2, g2, be2)


def kernel(votes_xyz, votes_feats, W1, b1, g1, be1, W2, b2, g2, be2):
    B, V, _ = votes_xyz.shape
    F = votes_feats.shape[2]
    xyz_t = jnp.transpose(votes_xyz, (0, 2, 1))

    nx, ny = _run_fps(xyz_t[:, 0, :], xyz_t[:, 1, :])
    idx, gx, gy = _run_ballquery(xyz_t, nx, ny)

    flat_idx = (idx + (jnp.arange(B, dtype=jnp.int32) * V)[:, None, None])
    flat_idx = flat_idx.reshape(B * _NPROP * _K)
    table = votes_feats.reshape(B * V, F)
    feats_g = _run_sc_gather(table, flat_idx)

    gxy = jnp.stack([gx.reshape(-1), gy.reshape(-1)], axis=1)
    w1f = jnp.transpose(W1[:, 2:])
    w1xy = jnp.transpose(W1[:, :2])
    w2t = jnp.transpose(W2)
    pooled = _run_mlp(feats_g, gxy, w1f, w1xy, b1[None, :], g1[None, :],
                      be1[None, :], w2t, b2[None, :], g2[None, :],
                      be2[None, :])

    new_xyz = jnp.stack(
        [nx[:, 0, :], ny[:, 0, :]], axis=-1)
    return new_xyz, pooled.reshape(B, _NPROP, -1)


# speedup vs baseline: 1.2228x; 1.2228x over previous
import functools

import jax
import jax.numpy as jnp
from jax import lax
from jax.experimental import pallas as pl
from jax.experimental.pallas import tpu as pltpu
from jax.experimental.pallas import tpu_sc as plsc

_NPROP = 128
_RADIUS = 0.3
_K = 16
_INF = float("inf")


def _fps_kernel(xs_ref, ys_ref, nx_ref, ny_ref):
    xv = xs_ref[...]
    yv = ys_ref[...]
    B, V = xv.shape
    colV = lax.broadcasted_iota(jnp.int32, (B, V), 1)
    colP = lax.broadcasted_iota(jnp.int32, (B, _NPROP), 1)

    def step(i, carry):
        dist, far, nx, ny = carry
        sel = colV == far
        cx = jnp.sum(jnp.where(sel, xv, 0.0), axis=1, keepdims=True)
        cy = jnp.sum(jnp.where(sel, yv, 0.0), axis=1, keepdims=True)
        emit = colP == i
        nx = jnp.where(emit, cx, nx)
        ny = jnp.where(emit, cy, ny)
        d = (xv - cx) ** 2 + (yv - cy) ** 2
        dist = jnp.minimum(dist, d)
        far = jnp.argmax(dist, axis=1).astype(jnp.int32)[:, None]
        return dist, far, nx, ny

    dist0 = jnp.full((B, V), 1e10, dtype=jnp.float32)
    zeros = jnp.zeros((B, _NPROP), dtype=jnp.float32)
    _, _, nx, ny = lax.fori_loop(
        0, _NPROP, step, (dist0, jnp.zeros((B, 1), jnp.int32), zeros, zeros))
    nx_ref[...] = nx[:, None, :]
    ny_ref[...] = ny[:, None, :]


def _run_fps(xs, ys):
    B, V = xs.shape
    return pl.pallas_call(
        _fps_kernel,
        out_shape=[
            jax.ShapeDtypeStruct((B, 1, _NPROP), jnp.float32),
            jax.ShapeDtypeStruct((B, 1, _NPROP), jnp.float32),
        ],
    )(xs, ys)


def _bq_kernel(xyz_ref, nx_ref, ny_ref, idx_ref, gx_ref, gy_ref):
    xv = xyz_ref[0, 0:1, :]
    yv = xyz_ref[0, 1:2, :]
    V = xv.shape[1]
    nx = jnp.transpose(nx_ref[0])
    ny = jnp.transpose(ny_ref[0])
    d = jnp.sqrt((nx - xv) ** 2 + (ny - yv) ** 2)
    col = lax.broadcasted_iota(jnp.int32, (_NPROP, V), 1)
    x0 = jnp.sum(xv[:, 0:1])
    y0 = jnp.sum(yv[:, 0:1])

    nxr = nx[:, 0]
    nyr = ny[:, 0]
    for k in range(_K):
        idxk = jnp.argmin(d, axis=1).astype(jnp.int32)
        onehot = col == idxk[:, None]
        xk = jnp.sum(jnp.where(onehot, xv, 0.0), axis=1)
        yk = jnp.sum(jnp.where(onehot, yv, 0.0), axis=1)
        d = jnp.where(onehot, _INF, d)
        dsel = jnp.sqrt((nxr - xk) ** 2 + (nyr - yk) ** 2)
        inr = dsel <= _RADIUS
        idx_ref[0, :, k] = jnp.where(inr, idxk, 0)
        gx_ref[0, :, k] = jnp.where(inr, xk, x0) - nxr
        gy_ref[0, :, k] = jnp.where(inr, yk, y0) - nyr


def _run_ballquery(xyz_t, nx, ny):
    B, _, V = xyz_t.shape
    return pl.pallas_call(
        _bq_kernel,
        grid=(B,),
        in_specs=[
            pl.BlockSpec((1, 2, V), lambda b: (b, 0, 0)),
            pl.BlockSpec((1, 1, _NPROP), lambda b: (b, 0, 0)),
            pl.BlockSpec((1, 1, _NPROP), lambda b: (b, 0, 0)),
        ],
        out_specs=[
            pl.BlockSpec((1, _NPROP, _K), lambda b: (b, 0, 0)),
            pl.BlockSpec((1, _NPROP, _K), lambda b: (b, 0, 0)),
            pl.BlockSpec((1, _NPROP, _K), lambda b: (b, 0, 0)),
        ],
        out_shape=[
            jax.ShapeDtypeStruct((B, _NPROP, _K), jnp.int32),
            jax.ShapeDtypeStruct((B, _NPROP, _K), jnp.float32),
            jax.ShapeDtypeStruct((B, _NPROP, _K), jnp.float32),
        ],
    )(xyz_t, nx, ny)


@functools.lru_cache(maxsize=None)
def _make_sc_gather(n_rows, feat):
    info = plsc.get_sparse_core_info()
    nw = info.num_cores * info.num_subcores
    rows_per_w = n_rows // nw
    mesh = plsc.VectorSubcoreMesh(core_axis_name="c", subcore_axis_name="s")

    @functools.partial(
        pl.kernel,
        mesh=mesh,
        out_type=jax.ShapeDtypeStruct((n_rows, feat), jnp.float32),
        scratch_types=[
            pltpu.VMEM((rows_per_w,), jnp.int32),
            pltpu.VMEM((rows_per_w, feat), jnp.float32),
            pltpu.SemaphoreType.DMA,
        ],
    )
    def gather_k(table_hbm, idx_hbm, out_hbm, idx_v, rows_v, sem):
        wid = lax.axis_index("s") * info.num_cores + lax.axis_index("c")
        base = wid * rows_per_w
        pltpu.sync_copy(idx_hbm.at[pl.ds(base, rows_per_w)], idx_v)
        pltpu.async_copy(table_hbm.at[idx_v], rows_v, sem).wait()
        pltpu.sync_copy(rows_v, out_hbm.at[pl.ds(base, rows_per_w)])

    return gather_k


def _run_sc_gather(table, idx_flat):
    return _make_sc_gather(idx_flat.shape[0], table.shape[1])(table, idx_flat)


def _mlp_kernel(feats_ref, gxy_ref, w1f_ref, w1xy_ref, b1_ref, g1_ref,
                be1_ref, w2_ref, b2_ref, g2_ref, be2_ref, out_ref):
    n = feats_ref.shape[0]
    h1 = (jnp.dot(feats_ref[...], w1f_ref[...],
                  preferred_element_type=jnp.float32)
          + jnp.dot(gxy_ref[...], w1xy_ref[...],
                    preferred_element_type=jnp.float32)
          + b1_ref[...])
    mu = jnp.mean(h1, axis=0, keepdims=True)
    var = jnp.mean((h1 - mu) ** 2, axis=0, keepdims=True)
    a1 = (h1 - mu) * lax.rsqrt(var + 1e-5) * g1_ref[...] + be1_ref[...]
    a1 = jnp.maximum(a1, 0.0)
    h2 = jnp.dot(a1, w2_ref[...], preferred_element_type=jnp.float32) + b2_ref[...]
    mu2 = jnp.mean(h2, axis=0, keepdims=True)
    var2 = jnp.mean((h2 - mu2) ** 2, axis=0, keepdims=True)
    a2 = (h2 - mu2) * lax.rsqrt(var2 + 1e-5) * g2_ref[...] + be2_ref[...]
    a2 = jnp.maximum(a2, 0.0)
    pooled = jnp.max(a2.reshape(n // _K, _K, a2.shape[1]), axis=1)
    out_ref[...] = pooled


def _run_mlp(feats_g, gxy, w1f, w1xy, b1, g1, be1, w2t, b2, g2, be2):
    n, cf = feats_g.shape
    co = w2t.shape[1]
    return pl.pallas_call(
        _mlp_kernel,
        out_shape=jax.ShapeDtypeStruct((n // _K, co), jnp.float32),
    )(feats_g, gxy, w1f, w1xy, b1, g1, be1, w2t, b2, g2, be2)


def kernel(votes_xyz, votes_feats, W1, b1, g1, be1, W2, b2, g2, be2):
    B, V, _ = votes_xyz.shape
    F = votes_feats.shape[2]
    xyz_t = jnp.transpose(votes_xyz, (0, 2, 1))

    nx, ny = _run_fps(xyz_t[:, 0, :], xyz_t[:, 1, :])
    idx, gx, gy = _run_ballquery(xyz_t, nx, ny)

    flat_idx = (idx + (jnp.arange(B, dtype=jnp.int32) * V)[:, None, None])
    flat_idx = flat_idx.reshape(B * _NPROP * _K)
    table = votes_feats.reshape(B * V, F)
    feats_g = _run_sc_gather(table, flat_idx)

    gxy = jnp.stack([gx.reshape(-1), gy.reshape(-1)], axis=1)
    w1f = jnp.transpose(W1[:, 2:])
    w1xy = jnp.transpose(W1[:, :2])
    w2t = jnp.transpose(W2)
    pooled = _run_mlp(feats_g, gxy, w1f, w1xy, b1[None, :], g1[None, :],
                      be1[None, :], w2t, b2[None, :], g2[None, :],
                      be2[None, :])

    new_xyz = jnp.stack(
        [nx[:, 0, :], ny[:, 0, :]], axis=-1)
    return new_xyz, pooled.reshape(B, _NPROP, -1)
